# baseline (speedup 1.0000x reference)
import functools

import jax
import jax.numpy as jnp
from jax import lax
from jax.experimental import pallas as pl
from jax.experimental.pallas import tpu as pltpu
from jax.experimental.pallas import tpu_sc as plsc

GAUSS_END = 5.5

_SC_CORES = 2
_SC_SUBCORES = 16
_NW = _SC_CORES * _SC_SUBCORES
_CHUNK = 128


def _sc_gather(table, idx):
    bsz = idx.shape[0]
    dim = table.shape[1]
    assert bsz % (8 * _NW) == 0, bsz
    bpw = bsz // _NW
    full = bpw // _CHUNK
    rem = bpw % _CHUNK
    assert rem % 8 == 0, rem
    full_p = full if full % 2 == 1 else full - 1
    mesh = plsc.VectorSubcoreMesh(core_axis_name="c", subcore_axis_name="s")

    @functools.partial(
        pl.kernel,
        out_type=jax.ShapeDtypeStruct((bsz, dim), table.dtype),
        mesh=mesh,
        scratch_types=[
            pltpu.VMEM((bpw,), jnp.int32),
            pltpu.VMEM((_CHUNK, dim), table.dtype),
            pltpu.VMEM((_CHUNK, dim), table.dtype),
            pltpu.SemaphoreType.DMA,
            pltpu.SemaphoreType.DMA,
        ],
    )
    def gather_kernel(table_hbm, idx_hbm, out_hbm, idx_v, r0, r1, s0, s1):
        wid = lax.axis_index("s") * _SC_CORES + lax.axis_index("c")
        base = wid * bpw
        pltpu.sync_copy(idx_hbm.at[pl.ds(base, bpw)], idx_v)

        def start(c, rv, sem):
            iv = idx_v.at[pl.ds(c * _CHUNK, _CHUNK)]
            pltpu.async_copy(table_hbm.at[iv], rv, sem)

        def drain(c, rv, sem):
            pltpu.make_async_copy(table_hbm.at[idx_v.at[pl.ds(0, _CHUNK)]],
                                  rv, sem).wait()
            pltpu.sync_copy(rv, out_hbm.at[pl.ds(base + c * _CHUNK, _CHUNK)])

        if full_p >= 1:
            start(0, r0, s0)

            def pair(p, carry):
                start(2 * p + 1, r1, s1)
                drain(2 * p, r0, s0)
                start(2 * p + 2, r0, s0)
                drain(2 * p + 1, r1, s1)
                return carry

            lax.fori_loop(0, (full_p - 1) // 2, pair, 0)
            drain(full_p - 1, r0, s0)
        if full_p < full:
            c = full_p
            start(c, r1, s1)
            drain(c, r1, s1)
        if rem:
            iv = idx_v.at[pl.ds(full * _CHUNK, rem)]
            rv = r0.at[pl.ds(0, rem)]
            pltpu.async_copy(table_hbm.at[iv], rv, s0).wait()
            pltpu.sync_copy(rv, out_hbm.at[pl.ds(base + full * _CHUNK, rem)])

    return gather_kernel(table, idx)


_LOG2E = 1.4426950408889634
_LN2 = 0.6931471805599453


def _neg_abs(x):
    xi = lax.bitcast_convert_type(x, jnp.int32)
    return lax.bitcast_convert_type(
        jnp.bitwise_or(xi, jnp.int32(-2147483648)), jnp.float32)


def _gate(v):
    return 1.0 / (1.0 + jnp.exp2(v))


def _core(u):
    return jnp.maximum(u, 0.0) + jnp.log2(1.0 + jnp.exp2(_neg_abs(u))) - 1.0


_T = 400


def _e0_expand(dist_t, fn, nbr):
    at = dist_t.shape[1]
    grid = at // _T

    def body(d_ref, e_ref):
        d = d_ref[...]
        inv_w = (fn - 1) / GAUSS_END
        offs = lax.broadcasted_iota(
            jnp.int32, (1, 1, fn), 2).astype(jnp.float32)
        t = d * inv_w - offs
        e_ref[...] = jnp.exp2((-0.5 * _LOG2E) * t * t).astype(jnp.bfloat16)

    return pl.pallas_call(
        body,
        grid=(grid,),
        in_specs=[pl.BlockSpec((nbr, _T, 1), lambda i: (0, i, 0))],
        out_specs=[pl.BlockSpec((nbr, _T, fn), lambda i: (0, i, 0))],
        out_shape=[jax.ShapeDtypeStruct((nbr, at, fn), jnp.bfloat16)],
    )(dist_t)[0]


def _part(h, nbr_h, e_in, mask_t, w, b, fn, nbr, out_h):
    at = h.shape[0]
    grid = at // _T
    te = nbr * _T

    def body(h_ref, nbr_ref, e_ref, m_ref, w_ref, b_ref, o_ref):
        i = pl.program_id(0)
        h_tile = h_ref[pl.ds(i * _T, _T), :]
        eb = e_ref[...]
        nbrv = nbr_ref[...]
        z_hi = jnp.dot(h_tile.astype(jnp.bfloat16), w_ref[0:fn, :],
                       preferred_element_type=jnp.float32) + b_ref[0]
        z = jnp.dot(nbrv.reshape(te, fn).astype(jnp.bfloat16),
                    w_ref[fn:2 * fn, :],
                    preferred_element_type=jnp.float32)
        z = z + jnp.dot(eb.reshape(te, fn), w_ref[2 * fn:3 * fn, :],
                        preferred_element_type=jnp.float32)
        z = z.reshape(nbr, _T, 2 * fn) + z_hi[None]
        m = m_ref[...] * _LN2
        prod = _gate(z[:, :, 0:fn]) * _core(z[:, :, fn:2 * fn]) * m
        if out_h:
            o_ref[...] = h_tile + prod.sum(axis=0)
        else:
            o_ref[...] = (eb.astype(jnp.float32) + prod).astype(jnp.bfloat16)

    if out_h:
        out_spec = pl.BlockSpec((_T, fn), lambda i: (i, 0))
        out_shape = jax.ShapeDtypeStruct((at, fn), jnp.float32)
    else:
        out_spec = pl.BlockSpec((nbr, _T, fn), lambda i: (0, i, 0))
        out_shape = jax.ShapeDtypeStruct((nbr, at, fn), jnp.bfloat16)

    return pl.pallas_call(
        body,
        grid=(grid,),
        in_specs=[
            pl.BlockSpec((at, fn), lambda i: (0, 0)),
            pl.BlockSpec((nbr, _T, fn), lambda i: (0, i, 0)),
            pl.BlockSpec((nbr, _T, fn), lambda i: (0, i, 0)),
            pl.BlockSpec((nbr, _T, 1), lambda i: (0, i, 0)),
            pl.BlockSpec(w.shape, lambda i: (0, 0)),
            pl.BlockSpec(b.shape, lambda i: (0, 0)),
        ],
        out_specs=[out_spec],
        out_shape=[out_shape],
    )(h, nbr_h, e_in, mask_t, w, b)[0]


def _last_layer(h, nbr_h, e_in, mask_t, uv, w, b, w1, b1, w2, b2, fn, nbr):
    at = h.shape[0]
    grid = at // _T
    te = nbr * _T

    def body(h_ref, nbr_ref, e_ref, m_ref, uv_ref, w_ref, b_ref,
             w1_ref, b1_ref, w2_ref, b2_ref, f_ref):
        i = pl.program_id(0)
        h_tile = h_ref[pl.ds(i * _T, _T), :]
        eb = e_ref[...]
        nbrv = nbr_ref[...]
        z_hi = jnp.dot(h_tile.astype(jnp.bfloat16), w_ref[0:fn, :],
                       preferred_element_type=jnp.float32) + b_ref[0]
        z = jnp.dot(nbrv.reshape(te, fn).astype(jnp.bfloat16),
                    w_ref[fn:2 * fn, :],
                    preferred_element_type=jnp.float32)
        z = z + jnp.dot(eb.reshape(te, fn), w_ref[2 * fn:3 * fn, :],
                        preferred_element_type=jnp.float32)
        z = z.reshape(nbr, _T, 2 * fn) + z_hi[None]
        m = m_ref[...]
        m2 = m * _LN2
        gate_e = _gate(z[:, :, 0:fn])
        core_e = _core(z[:, :, fn:2 * fn])
        e3 = eb.astype(jnp.float32) + gate_e * core_e * m2
        x = _core(jnp.dot(e3.astype(jnp.bfloat16).reshape(te, fn), w1_ref[...],
                          preferred_element_type=jnp.float32) + b1_ref[...])
        s = jnp.dot(x.astype(jnp.bfloat16), w2_ref[...],
                    preferred_element_type=jnp.float32) + b2_ref[...]
        f = s.reshape(nbr, _T, 1) * uv_ref[...] * m
        f_ref[...] = f.sum(axis=0)

    return pl.pallas_call(
        body,
        grid=(grid,),
        in_specs=[
            pl.BlockSpec((at, fn), lambda i: (0, 0)),
            pl.BlockSpec((nbr, _T, fn), lambda i: (0, i, 0)),
            pl.BlockSpec((nbr, _T, fn), lambda i: (0, i, 0)),
            pl.BlockSpec((nbr, _T, 1), lambda i: (0, i, 0)),
            pl.BlockSpec((nbr, _T, 3), lambda i: (0, i, 0)),
            pl.BlockSpec(w.shape, lambda i: (0, 0)),
            pl.BlockSpec(b.shape, lambda i: (0, 0)),
            pl.BlockSpec(w1.shape, lambda i: (0, 0)),
            pl.BlockSpec(b1.shape, lambda i: (0, 0)),
            pl.BlockSpec(w2.shape, lambda i: (0, 0)),
            pl.BlockSpec(b2.shape, lambda i: (0, 0)),
        ],
        out_specs=[pl.BlockSpec((_T, 3), lambda i: (i, 0))],
        out_shape=[jax.ShapeDtypeStruct((at, 3), jnp.float32)],
    )(h, nbr_h, e_in, mask_t, uv, w, b, w1, b1, w2, b2)[0]


def kernel(atomic_numbers, neighbors, neighbor_mask, distances, unit_vecs, params):
    b, at, nbr = neighbors.shape
    fn = params['emb_table'].shape[1]

    an_flat = atomic_numbers.reshape(-1).astype(jnp.int32)
    pad = (-an_flat.shape[0]) % (8 * _NW)
    an_pad = jnp.pad(an_flat, (0, pad))
    idx_n = jnp.transpose(neighbors.reshape(at, nbr)).reshape(-1).astype(jnp.int32)
    mask_t = jnp.transpose(neighbor_mask.reshape(at, nbr)).reshape(nbr, at, 1)
    dist_t = jnp.transpose(distances.reshape(at, nbr)).reshape(nbr, at, 1)
    uv_t = jnp.transpose(unit_vecs.reshape(at, nbr, 3), (1, 0, 2))

    layers = params['layers']
    gl, cl = -_LOG2E, _LOG2E

    def stack(lyr, f, s):
        w = jnp.concatenate([gl * lyr[f[0]], cl * lyr[s[0]]], axis=1)
        bias = jnp.concatenate([gl * lyr[f[1]], cl * lyr[s[1]]])
        return w.astype(jnp.bfloat16), bias.reshape(1, -1)

    wn = [stack(l, ('Wf_n', 'bf_n'), ('Ws_n', 'bs_n')) for l in layers[:-1]]
    we = [stack(l, ('Wf_e', 'bf_e'), ('Ws_e', 'bs_e')) for l in layers]

    e = _e0_expand(dist_t, fn, nbr)
    h = _sc_gather(params['emb_table'], an_pad)[:at]
    nbr_h = _sc_gather(h, idx_n).reshape(nbr, at, fn)
    for li in range(len(layers) - 1):
        h_new = _part(h, nbr_h, e, mask_t, *wn[li], fn, nbr, True)
        nbr_h_next = _sc_gather(h_new, idx_n).reshape(nbr, at, fn)
        e = _part(h, nbr_h, e, mask_t, *we[li], fn, nbr, False)
        h, nbr_h = h_new, nbr_h_next
    forces = _last_layer(
        h, nbr_h, e, mask_t, uv_t, we[-1][0], we[-1][1],
        (_LOG2E * params['out_W1']).astype(jnp.bfloat16),
        _LOG2E * params['out_b1'].reshape(1, -1),
        (_LN2 * params['out_W2']).astype(jnp.bfloat16),
        params['out_b2'].reshape(1, -1),
        fn, nbr)
    return forces.reshape(b, at, 3)


# speedup vs baseline: 1.0141x; 1.0141x over previous
import functools

import jax
import jax.numpy as jnp
from jax import lax
from jax.experimental import pallas as pl
from jax.experimental.pallas import tpu as pltpu
from jax.experimental.pallas import tpu_sc as plsc

GAUSS_END = 5.5

_SC_CORES = 2
_SC_SUBCORES = 16
_NW = _SC_CORES * _SC_SUBCORES
_CHUNK = 128


def _sc_gather(table, idx):
    bsz = idx.shape[0]
    dim = table.shape[1]
    assert bsz % (8 * _NW) == 0, bsz
    bpw = bsz // _NW
    full = bpw // _CHUNK
    rem = bpw % _CHUNK
    assert rem % 8 == 0, rem
    full_p = full if full % 2 == 1 else full - 1
    mesh = plsc.VectorSubcoreMesh(core_axis_name="c", subcore_axis_name="s")

    @functools.partial(
        pl.kernel,
        out_type=jax.ShapeDtypeStruct((bsz, dim), table.dtype),
        mesh=mesh,
        scratch_types=[
            pltpu.VMEM((bpw,), jnp.int32),
            pltpu.VMEM((_CHUNK, dim), table.dtype),
            pltpu.VMEM((_CHUNK, dim), table.dtype),
            pltpu.SemaphoreType.DMA,
            pltpu.SemaphoreType.DMA,
        ],
    )
    def gather_kernel(table_hbm, idx_hbm, out_hbm, idx_v, r0, r1, s0, s1):
        wid = lax.axis_index("s") * _SC_CORES + lax.axis_index("c")
        base = wid * bpw
        pltpu.sync_copy(idx_hbm.at[pl.ds(base, bpw)], idx_v)

        def start(c, rv, sem):
            iv = idx_v.at[pl.ds(c * _CHUNK, _CHUNK)]
            pltpu.async_copy(table_hbm.at[iv], rv, sem)

        def drain(c, rv, sem):
            pltpu.make_async_copy(table_hbm.at[idx_v.at[pl.ds(0, _CHUNK)]],
                                  rv, sem).wait()
            pltpu.sync_copy(rv, out_hbm.at[pl.ds(base + c * _CHUNK, _CHUNK)])

        if full_p >= 1:
            start(0, r0, s0)

            def pair(p, carry):
                start(2 * p + 1, r1, s1)
                drain(2 * p, r0, s0)
                start(2 * p + 2, r0, s0)
                drain(2 * p + 1, r1, s1)
                return carry

            lax.fori_loop(0, (full_p - 1) // 2, pair, 0)
            drain(full_p - 1, r0, s0)
        if full_p < full:
            c = full_p
            start(c, r1, s1)
            drain(c, r1, s1)
        if rem:
            iv = idx_v.at[pl.ds(full * _CHUNK, rem)]
            rv = r0.at[pl.ds(0, rem)]
            pltpu.async_copy(table_hbm.at[iv], rv, s0).wait()
            pltpu.sync_copy(rv, out_hbm.at[pl.ds(base + full * _CHUNK, rem)])

    return gather_kernel(table, idx)


_LOG2E = 1.4426950408889634
_LN2 = 0.6931471805599453


def _neg_abs(x):
    xi = lax.bitcast_convert_type(x, jnp.int32)
    return lax.bitcast_convert_type(
        jnp.bitwise_or(xi, jnp.int32(-2147483648)), jnp.float32)


def _gate(v):
    return 1.0 / (1.0 + jnp.exp2(v))


def _core(u):
    return jnp.maximum(u, 0.0) + jnp.log2(1.0 + jnp.exp2(_neg_abs(u))) - 1.0


_T = 400


def _e0_expand(dist_t, fn, nbr):
    at = dist_t.shape[1]
    grid = at // _T

    def body(d_ref, e_ref):
        d = d_ref[...]
        inv_w = (fn - 1) / GAUSS_END
        offs = lax.broadcasted_iota(
            jnp.int32, (1, 1, fn), 2).astype(jnp.float32)
        t = d * inv_w - offs
        e_ref[...] = jnp.exp2((-0.5 * _LOG2E) * t * t).astype(jnp.bfloat16)

    return pl.pallas_call(
        body,
        grid=(grid,),
        in_specs=[pl.BlockSpec((nbr, _T, 1), lambda i: (0, i, 0))],
        out_specs=[pl.BlockSpec((nbr, _T, fn), lambda i: (0, i, 0))],
        out_shape=[jax.ShapeDtypeStruct((nbr, at, fn), jnp.bfloat16)],
    )(dist_t)[0]


def _mp_layer(h, nbr_h, e_in, mask_t, w, b, fn, nbr):
    at = h.shape[0]
    grid = at // _T
    te = nbr * _T

    def body(h_ref, nbr_ref, e_ref, m_ref, w_ref, b_ref, ho_ref, eo_ref):
        i = pl.program_id(0)
        h_tile = h_ref[pl.ds(i * _T, _T), :]
        eb = e_ref[...]
        nbrv = nbr_ref[...]
        z_hi = jnp.dot(h_tile.astype(jnp.bfloat16), w_ref[0:fn, :],
                       preferred_element_type=jnp.float32) + b_ref[0]
        z = jnp.dot(nbrv.reshape(te, fn).astype(jnp.bfloat16),
                    w_ref[fn:2 * fn, :],
                    preferred_element_type=jnp.float32)
        z = z + jnp.dot(eb.reshape(te, fn), w_ref[2 * fn:3 * fn, :],
                        preferred_element_type=jnp.float32)
        z = z.reshape(nbr, _T, 4 * fn) + z_hi[None]
        m = m_ref[...] * _LN2
        pn = _gate(z[:, :, 0:fn]) * _core(z[:, :, fn:2 * fn]) * m
        pe = _gate(z[:, :, 2 * fn:3 * fn]) * _core(z[:, :, 3 * fn:4 * fn]) * m
        ho_ref[...] = h_tile + pn.sum(axis=0)
        eo_ref[...] = (eb.astype(jnp.float32) + pe).astype(jnp.bfloat16)

    return pl.pallas_call(
        body,
        grid=(grid,),
        in_specs=[
            pl.BlockSpec((at, fn), lambda i: (0, 0)),
            pl.BlockSpec((nbr, _T, fn), lambda i: (0, i, 0)),
            pl.BlockSpec((nbr, _T, fn), lambda i: (0, i, 0)),
            pl.BlockSpec((nbr, _T, 1), lambda i: (0, i, 0)),
            pl.BlockSpec(w.shape, lambda i: (0, 0)),
            pl.BlockSpec(b.shape, lambda i: (0, 0)),
        ],
        out_specs=[
            pl.BlockSpec((_T, fn), lambda i: (i, 0)),
            pl.BlockSpec((nbr, _T, fn), lambda i: (0, i, 0)),
        ],
        out_shape=[
            jax.ShapeDtypeStruct((at, fn), jnp.float32),
            jax.ShapeDtypeStruct((nbr, at, fn), jnp.bfloat16),
        ],
    )(h, nbr_h, e_in, mask_t, w, b)


def _last_layer(h, nbr_h, e_in, mask_t, uv, w, b, w1, b1, w2, b2, fn, nbr):
    at = h.shape[0]
    grid = at // _T
    te = nbr * _T

    def body(h_ref, nbr_ref, e_ref, m_ref, uv_ref, w_ref, b_ref,
             w1_ref, b1_ref, w2_ref, b2_ref, f_ref):
        i = pl.program_id(0)
        h_tile = h_ref[pl.ds(i * _T, _T), :]
        eb = e_ref[...]
        nbrv = nbr_ref[...]
        z_hi = jnp.dot(h_tile.astype(jnp.bfloat16), w_ref[0:fn, :],
                       preferred_element_type=jnp.float32) + b_ref[0]
        z = jnp.dot(nbrv.reshape(te, fn).astype(jnp.bfloat16),
                    w_ref[fn:2 * fn, :],
                    preferred_element_type=jnp.float32)
        z = z + jnp.dot(eb.reshape(te, fn), w_ref[2 * fn:3 * fn, :],
                        preferred_element_type=jnp.float32)
        z = z.reshape(nbr, _T, 2 * fn) + z_hi[None]
        m = m_ref[...]
        m2 = m * _LN2
        gate_e = _gate(z[:, :, 0:fn])
        core_e = _core(z[:, :, fn:2 * fn])
        e3 = eb.astype(jnp.float32) + gate_e * core_e * m2
        x = _core(jnp.dot(e3.astype(jnp.bfloat16).reshape(te, fn), w1_ref[...],
                          preferred_element_type=jnp.float32) + b1_ref[...])
        s = jnp.dot(x.astype(jnp.bfloat16), w2_ref[...],
                    preferred_element_type=jnp.float32) + b2_ref[...]
        f = s.reshape(nbr, _T, 1) * uv_ref[...] * m
        f_ref[...] = f.sum(axis=0)

    return pl.pallas_call(
        body,
        grid=(grid,),
        in_specs=[
            pl.BlockSpec((at, fn), lambda i: (0, 0)),
            pl.BlockSpec((nbr, _T, fn), lambda i: (0, i, 0)),
            pl.BlockSpec((nbr, _T, fn), lambda i: (0, i, 0)),
            pl.BlockSpec((nbr, _T, 1), lambda i: (0, i, 0)),
            pl.BlockSpec((nbr, _T, 3), lambda i: (0, i, 0)),
            pl.BlockSpec(w.shape, lambda i: (0, 0)),
            pl.BlockSpec(b.shape, lambda i: (0, 0)),
            pl.BlockSpec(w1.shape, lambda i: (0, 0)),
            pl.BlockSpec(b1.shape, lambda i: (0, 0)),
            pl.BlockSpec(w2.shape, lambda i: (0, 0)),
            pl.BlockSpec(b2.shape, lambda i: (0, 0)),
        ],
        out_specs=[pl.BlockSpec((_T, 3), lambda i: (i, 0))],
        out_shape=[jax.ShapeDtypeStruct((at, 3), jnp.float32)],
    )(h, nbr_h, e_in, mask_t, uv, w, b, w1, b1, w2, b2)[0]


def kernel(atomic_numbers, neighbors, neighbor_mask, distances, unit_vecs, params):
    b, at, nbr = neighbors.shape
    fn = params['emb_table'].shape[1]

    an_flat = atomic_numbers.reshape(-1).astype(jnp.int32)
    pad = (-an_flat.shape[0]) % (8 * _NW)
    an_pad = jnp.pad(an_flat, (0, pad))
    idx_n = jnp.transpose(neighbors.reshape(at, nbr)).reshape(-1).astype(jnp.int32)
    mask_t = jnp.transpose(neighbor_mask.reshape(at, nbr)).reshape(nbr, at, 1)
    dist_t = jnp.transpose(distances.reshape(at, nbr)).reshape(nbr, at, 1)
    uv_t = jnp.transpose(unit_vecs.reshape(at, nbr, 3), (1, 0, 2))

    layers = params['layers']
    gl, cl = -_LOG2E, _LOG2E

    def stack(lyr, f, s):
        w = jnp.concatenate([gl * lyr[f[0]], cl * lyr[s[0]]], axis=1)
        bias = jnp.concatenate([gl * lyr[f[1]], cl * lyr[s[1]]])
        return w.astype(jnp.bfloat16), bias.reshape(1, -1)

    def stack4(lyr):
        w = jnp.concatenate(
            [gl * lyr['Wf_n'], cl * lyr['Ws_n'],
             gl * lyr['Wf_e'], cl * lyr['Ws_e']], axis=1)
        bias = jnp.concatenate(
            [gl * lyr['bf_n'], cl * lyr['bs_n'],
             gl * lyr['bf_e'], cl * lyr['bs_e']])
        return w.astype(jnp.bfloat16), bias.reshape(1, -1)

    wl = [stack4(l) for l in layers[:-1]]
    we = stack(layers[-1], ('Wf_e', 'bf_e'), ('Ws_e', 'bs_e'))

    e = _e0_expand(dist_t, fn, nbr)
    h = _sc_gather(params['emb_table'], an_pad)[:at]
    for li in range(len(layers) - 1):
        nbr_h = _sc_gather(h, idx_n).reshape(nbr, at, fn)
        h, e = _mp_layer(h, nbr_h, e, mask_t, *wl[li], fn, nbr)
    nbr_h = _sc_gather(h, idx_n).reshape(nbr, at, fn)
    forces = _last_layer(
        h, nbr_h, e, mask_t, uv_t, we[0], we[1],
        (_LOG2E * params['out_W1']).astype(jnp.bfloat16),
        _LOG2E * params['out_b1'].reshape(1, -1),
        (_LN2 * params['out_W2']).astype(jnp.bfloat16),
        params['out_b2'].reshape(1, -1),
        fn, nbr)
    return forces.reshape(b, at, 3)


# speedup vs baseline: 1.0785x; 1.0635x over previous
import functools

import jax
import jax.numpy as jnp
from jax import lax
from jax.experimental import pallas as pl
from jax.experimental.pallas import tpu as pltpu
from jax.experimental.pallas import tpu_sc as plsc

GAUSS_END = 5.5

_SC_CORES = 2
_SC_SUBCORES = 16
_NW = _SC_CORES * _SC_SUBCORES
_CHUNK = 128


def _sc_gather(table, idx):
    bsz = idx.shape[0]
    dim = table.shape[1]
    assert bsz % (8 * _NW) == 0, bsz
    bpw = bsz // _NW
    full = bpw // _CHUNK
    rem = bpw % _CHUNK
    assert rem % 8 == 0, rem
    full_p = full if full % 2 == 1 else full - 1
    mesh = plsc.VectorSubcoreMesh(core_axis_name="c", subcore_axis_name="s")

    @functools.partial(
        pl.kernel,
        out_type=jax.ShapeDtypeStruct((bsz, dim), table.dtype),
        mesh=mesh,
        scratch_types=[
            pltpu.VMEM((bpw,), jnp.int32),
            pltpu.VMEM((_CHUNK, dim), table.dtype),
            pltpu.VMEM((_CHUNK, dim), table.dtype),
            pltpu.SemaphoreType.DMA,
            pltpu.SemaphoreType.DMA,
        ],
    )
    def gather_kernel(table_hbm, idx_hbm, out_hbm, idx_v, r0, r1, s0, s1):
        wid = lax.axis_index("s") * _SC_CORES + lax.axis_index("c")
        base = wid * bpw
        pltpu.sync_copy(idx_hbm.at[pl.ds(base, bpw)], idx_v)

        def start(c, rv, sem):
            iv = idx_v.at[pl.ds(c * _CHUNK, _CHUNK)]
            pltpu.async_copy(table_hbm.at[iv], rv, sem)

        def drain(c, rv, sem):
            pltpu.make_async_copy(table_hbm.at[idx_v.at[pl.ds(0, _CHUNK)]],
                                  rv, sem).wait()
            pltpu.sync_copy(rv, out_hbm.at[pl.ds(base + c * _CHUNK, _CHUNK)])

        if full_p >= 1:
            start(0, r0, s0)

            def pair(p, carry):
                start(2 * p + 1, r1, s1)
                drain(2 * p, r0, s0)
                start(2 * p + 2, r0, s0)
                drain(2 * p + 1, r1, s1)
                return carry

            lax.fori_loop(0, (full_p - 1) // 2, pair, 0)
            drain(full_p - 1, r0, s0)
        if full_p < full:
            c = full_p
            start(c, r1, s1)
            drain(c, r1, s1)
        if rem:
            iv = idx_v.at[pl.ds(full * _CHUNK, rem)]
            rv = r0.at[pl.ds(0, rem)]
            pltpu.async_copy(table_hbm.at[iv], rv, s0).wait()
            pltpu.sync_copy(rv, out_hbm.at[pl.ds(base + full * _CHUNK, rem)])

    return gather_kernel(table, idx)


_LOG2E = 1.4426950408889634
_LN2 = 0.6931471805599453


def _neg_abs(x):
    xi = lax.bitcast_convert_type(x, jnp.int32)
    return lax.bitcast_convert_type(
        jnp.bitwise_or(xi, jnp.int32(-2147483648)), jnp.float32)


def _gate(v):
    return 1.0 / (1.0 + jnp.exp2(v))


def _core(u):
    return jnp.maximum(u, 0.0) + jnp.log2(1.0 + jnp.exp2(_neg_abs(u))) - 1.0


_T = 400


def _mp_layer(h, nbr_h, e_in, mask_t, w, b, fn, nbr):
    at = h.shape[0]
    grid = at // _T
    te = nbr * _T

    first = e_in.shape[2] == 1

    def body(h_ref, nbr_ref, e_ref, m_ref, w_ref, b_ref, ho_ref, eo_ref):
        i = pl.program_id(0)
        h_tile = h_ref[pl.ds(i * _T, _T), :]
        if first:
            d = e_ref[...]
            inv_w = (fn - 1) / GAUSS_END
            offs = lax.broadcasted_iota(
                jnp.int32, (1, 1, fn), 2).astype(jnp.float32)
            t = d * inv_w - offs
            ef = jnp.exp2((-0.5 * _LOG2E) * t * t)
            eb = ef.astype(jnp.bfloat16)
        else:
            eb = e_ref[...]
            ef = eb.astype(jnp.float32)
        nbrv = nbr_ref[...]
        z_hi = jnp.dot(h_tile.astype(jnp.bfloat16), w_ref[0:fn, :],
                       preferred_element_type=jnp.float32) + b_ref[0]
        z = jnp.dot(nbrv.reshape(te, fn).astype(jnp.bfloat16),
                    w_ref[fn:2 * fn, :],
                    preferred_element_type=jnp.float32)
        z = z + jnp.dot(eb.reshape(te, fn), w_ref[2 * fn:3 * fn, :],
                        preferred_element_type=jnp.float32)
        z = z.reshape(nbr, _T, 4 * fn) + z_hi[None]
        m = m_ref[...] * _LN2
        pn = _gate(z[:, :, 0:fn]) * _core(z[:, :, fn:2 * fn]) * m
        pe = _gate(z[:, :, 2 * fn:3 * fn]) * _core(z[:, :, 3 * fn:4 * fn]) * m
        ho_ref[...] = h_tile + pn.sum(axis=0)
        eo_ref[...] = (ef + pe).astype(jnp.bfloat16)

    return pl.pallas_call(
        body,
        grid=(grid,),
        in_specs=[
            pl.BlockSpec((at, fn), lambda i: (0, 0)),
            pl.BlockSpec((nbr, _T, fn), lambda i: (0, i, 0)),
            pl.BlockSpec((nbr, _T, e_in.shape[2]), lambda i: (0, i, 0)),
            pl.BlockSpec((nbr, _T, 1), lambda i: (0, i, 0)),
            pl.BlockSpec(w.shape, lambda i: (0, 0)),
            pl.BlockSpec(b.shape, lambda i: (0, 0)),
        ],
        out_specs=[
            pl.BlockSpec((_T, fn), lambda i: (i, 0)),
            pl.BlockSpec((nbr, _T, fn), lambda i: (0, i, 0)),
        ],
        out_shape=[
            jax.ShapeDtypeStruct((at, fn), jnp.float32),
            jax.ShapeDtypeStruct((nbr, at, fn), jnp.bfloat16),
        ],
    )(h, nbr_h, e_in, mask_t, w, b)


def _last_layer(h, nbr_h, e_in, mask_t, uv, w, b, w1, b1, w2, b2, fn, nbr):
    at = h.shape[0]
    grid = at // _T
    te = nbr * _T

    def body(h_ref, nbr_ref, e_ref, m_ref, uv_ref, w_ref, b_ref,
             w1_ref, b1_ref, w2_ref, b2_ref, f_ref):
        i = pl.program_id(0)
        h_tile = h_ref[pl.ds(i * _T, _T), :]
        eb = e_ref[...]
        nbrv = nbr_ref[...]
        z_hi = jnp.dot(h_tile.astype(jnp.bfloat16), w_ref[0:fn, :],
                       preferred_element_type=jnp.float32) + b_ref[0]
        z = jnp.dot(nbrv.reshape(te, fn).astype(jnp.bfloat16),
                    w_ref[fn:2 * fn, :],
                    preferred_element_type=jnp.float32)
        z = z + jnp.dot(eb.reshape(te, fn), w_ref[2 * fn:3 * fn, :],
                        preferred_element_type=jnp.float32)
        z = z.reshape(nbr, _T, 2 * fn) + z_hi[None]
        m = m_ref[...]
        m2 = m * _LN2
        gate_e = _gate(z[:, :, 0:fn])
        core_e = _core(z[:, :, fn:2 * fn])
        e3 = eb.astype(jnp.float32) + gate_e * core_e * m2
        x = _core(jnp.dot(e3.astype(jnp.bfloat16).reshape(te, fn), w1_ref[...],
                          preferred_element_type=jnp.float32) + b1_ref[...])
        s = jnp.dot(x.astype(jnp.bfloat16), w2_ref[...],
                    preferred_element_type=jnp.float32) + b2_ref[...]
        f = s.reshape(nbr, _T, 1) * uv_ref[...] * m
        f_ref[...] = f.sum(axis=0)

    return pl.pallas_call(
        body,
        grid=(grid,),
        in_specs=[
            pl.BlockSpec((at, fn), lambda i: (0, 0)),
            pl.BlockSpec((nbr, _T, fn), lambda i: (0, i, 0)),
            pl.BlockSpec((nbr, _T, fn), lambda i: (0, i, 0)),
            pl.BlockSpec((nbr, _T, 1), lambda i: (0, i, 0)),
            pl.BlockSpec((nbr, _T, 3), lambda i: (0, i, 0)),
            pl.BlockSpec(w.shape, lambda i: (0, 0)),
            pl.BlockSpec(b.shape, lambda i: (0, 0)),
            pl.BlockSpec(w1.shape, lambda i: (0, 0)),
            pl.BlockSpec(b1.shape, lambda i: (0, 0)),
            pl.BlockSpec(w2.shape, lambda i: (0, 0)),
            pl.BlockSpec(b2.shape, lambda i: (0, 0)),
        ],
        out_specs=[pl.BlockSpec((_T, 3), lambda i: (i, 0))],
        out_shape=[jax.ShapeDtypeStruct((at, 3), jnp.float32)],
    )(h, nbr_h, e_in, mask_t, uv, w, b, w1, b1, w2, b2)[0]


def kernel(atomic_numbers, neighbors, neighbor_mask, distances, unit_vecs, params):
    b, at, nbr = neighbors.shape
    fn = params['emb_table'].shape[1]

    an_flat = atomic_numbers.reshape(-1).astype(jnp.int32)
    pad = (-an_flat.shape[0]) % (8 * _NW)
    an_pad = jnp.pad(an_flat, (0, pad))
    idx_n = jnp.transpose(neighbors.reshape(at, nbr)).reshape(-1).astype(jnp.int32)
    mask_t = jnp.transpose(neighbor_mask.reshape(at, nbr)).reshape(nbr, at, 1)
    dist_t = jnp.transpose(distances.reshape(at, nbr)).reshape(nbr, at, 1)
    uv_t = jnp.transpose(unit_vecs.reshape(at, nbr, 3), (1, 0, 2))

    layers = params['layers']
    gl, cl = -_LOG2E, _LOG2E

    def stack(lyr, f, s):
        w = jnp.concatenate([gl * lyr[f[0]], cl * lyr[s[0]]], axis=1)
        bias = jnp.concatenate([gl * lyr[f[1]], cl * lyr[s[1]]])
        return w.astype(jnp.bfloat16), bias.reshape(1, -1)

    def stack4(lyr):
        w = jnp.concatenate(
            [gl * lyr['Wf_n'], cl * lyr['Ws_n'],
             gl * lyr['Wf_e'], cl * lyr['Ws_e']], axis=1)
        bias = jnp.concatenate(
            [gl * lyr['bf_n'], cl * lyr['bs_n'],
             gl * lyr['bf_e'], cl * lyr['bs_e']])
        return w.astype(jnp.bfloat16), bias.reshape(1, -1)

    wl = [stack4(l) for l in layers[:-1]]
    we = stack(layers[-1], ('Wf_e', 'bf_e'), ('Ws_e', 'bs_e'))

    e = dist_t
    h = _sc_gather(params['emb_table'], an_pad)[:at]
    for li in range(len(layers) - 1):
        nbr_h = _sc_gather(h, idx_n).reshape(nbr, at, fn)
        h, e = _mp_layer(h, nbr_h, e, mask_t, *wl[li], fn, nbr)
    nbr_h = _sc_gather(h, idx_n).reshape(nbr, at, fn)
    forces = _last_layer(
        h, nbr_h, e, mask_t, uv_t, we[0], we[1],
        (_LOG2E * params['out_W1']).astype(jnp.bfloat16),
        _LOG2E * params['out_b1'].reshape(1, -1),
        (_LN2 * params['out_W2']).astype(jnp.bfloat16),
        params['out_b2'].reshape(1, -1),
        fn, nbr)
    return forces.reshape(b, at, 3)
